# baseline (speedup 1.0000x reference)
import functools

import jax
import jax.numpy as jnp
from jax import lax
from jax.experimental import pallas as pl
from jax.experimental.pallas import tpu as pltpu
from jax.experimental.pallas import tpu_sc as plsc

_NC, _NS = 2, 16
_NW = _NC * _NS
_N = 10000
_E = 320000
_EP = 327680
_CH = 128
_NRI = _EP // _CH
_RPW = _NRI // _NW
_KG = 4
_NG = _RPW // _KG
_OUTR = _KG * _CH * 16 // 128
_EPK = _EP * 16 // 128
_E8 = _E * 16 // 128
_RPT = _EPK // _NW


def _sc_gather(ps_t, pr_t, sidx, ridx):
    npt = _N // _NS
    mesh = plsc.VectorSubcoreMesh(core_axis_name="c", subcore_axis_name="s")

    @functools.partial(
        pl.kernel,
        out_type=jax.ShapeDtypeStruct((_NRI, _CH, 16), jnp.float32),
        mesh=mesh,
        compiler_params=pltpu.CompilerParams(use_tc_tiling_on_sc=False),
        scratch_types=[
            pltpu.VMEM((_RPW, _CH), jnp.int32),
            pltpu.VMEM((_RPW, _CH), jnp.int32),
            pltpu.VMEM((_KG, _CH, 16), jnp.float32),
            pltpu.VMEM((_KG, _CH, 16), jnp.float32),
            pltpu.VMEM_SHARED((_N, 16), jnp.float32),
            pltpu.VMEM_SHARED((_N, 16), jnp.float32),
            pltpu.SemaphoreType.DMA,
            pltpu.SemaphoreType.DMA,
            pltpu.SemaphoreType.DMA,
            pltpu.SemaphoreType.DMA,
            pltpu.SemaphoreType.DMA,
            pltpu.SemaphoreType.DMA,
        ],
    )
    def k(ps_hbm, pr_hbm, sidx_hbm, ridx_hbm, out_hbm,
          idxs_v, idxr_v, buf_a, buf_b, tab_s, tab_r,
          sem_sa, sem_sb, sem_ra, sem_rb, sem_oa, sem_ob):
        sid = lax.axis_index("s")
        w = lax.axis_index("c") * _NS + sid
        pltpu.sync_copy(ps_hbm.at[pl.ds(sid * npt, npt)],
                        tab_s.at[pl.ds(sid * npt, npt)])
        pltpu.sync_copy(pr_hbm.at[pl.ds(sid * npt, npt)],
                        tab_r.at[pl.ds(sid * npt, npt)])
        pltpu.sync_copy(sidx_hbm.at[pl.ds(w * _RPW, _RPW)], idxs_v)
        pltpu.sync_copy(ridx_hbm.at[pl.ds(w * _RPW, _RPW)], idxr_v)
        plsc.subcore_barrier()
        bufs = (buf_a, buf_b)
        ssems, rsems, osems = (sem_sa, sem_sb), (sem_ra, sem_rb), (sem_oa, sem_ob)

        def s_fire(g):
            p = g % 2
            for j in range(_KG):
                pltpu.async_copy(
                    tab_s.at[idxs_v.at[g * _KG + j]], bufs[p].at[j], ssems[p])

        def s_drain(g):
            p = g % 2
            for j in range(_KG):
                pltpu.make_async_copy(
                    tab_s.at[idxs_v.at[g * _KG + j]], bufs[p].at[j],
                    ssems[p]).wait()

        def r_fire(g):
            p = g % 2
            for j in range(_KG):
                pltpu.async_copy(
                    tab_r.at[idxr_v.at[g * _KG + j]], bufs[p].at[j], rsems[p],
                    add=True)

        def r_drain(g):
            p = g % 2
            for j in range(_KG):
                pltpu.make_async_copy(
                    tab_r.at[idxr_v.at[g * _KG + j]], bufs[p].at[j],
                    rsems[p]).wait()

        def o_fire(g):
            p = g % 2
            pltpu.async_copy(
                bufs[p], out_hbm.at[pl.ds(w * _RPW + g * _KG, _KG)], osems[p])

        def o_drain(g):
            p = g % 2
            pltpu.make_async_copy(
                bufs[p], out_hbm.at[pl.ds(w * _RPW + g * _KG, _KG)],
                osems[p]).wait()

        s_fire(0)
        for g in range(_NG):
            s_drain(g)
            r_fire(g)
            if g + 1 < _NG:
                if g >= 1:
                    o_drain(g - 1)
                s_fire(g + 1)
            r_drain(g)
            o_fire(g)
        o_drain(_NG - 2)
        o_drain(_NG - 1)

    return k(ps_t, pr_t, sidx, ridx)


def _sc_scatter(ne, sidx, ridx):
    mesh = plsc.VectorSubcoreMesh(core_axis_name="c", subcore_axis_name="s")

    @functools.partial(
        pl.kernel,
        out_type=(
            jax.ShapeDtypeStruct((_NC, _N, 16), jnp.float32),
            jax.ShapeDtypeStruct((_NC, _N, 16), jnp.float32),
        ),
        mesh=mesh,
        compiler_params=pltpu.CompilerParams(use_tc_tiling_on_sc=False),
        scratch_types=[
            pltpu.VMEM((_RPW, _CH), jnp.int32),
            pltpu.VMEM((_RPW, _CH), jnp.int32),
            pltpu.VMEM((_KG, _CH, 16), jnp.float32),
            pltpu.VMEM((_KG, _CH, 16), jnp.float32),
            pltpu.VMEM((625, 16), jnp.float32),
            pltpu.VMEM_SHARED((_N, 16), jnp.float32),
            pltpu.VMEM_SHARED((_N, 16), jnp.float32),
            pltpu.SemaphoreType.DMA,
            pltpu.SemaphoreType.DMA,
            pltpu.SemaphoreType.DMA,
            pltpu.SemaphoreType.DMA,
        ],
    )
    def k(ne_hbm, sidx_hbm, ridx_hbm, sagg_hbm, ragg_hbm,
          idxs_v, idxr_v, buf_a, buf_b, zbuf, shr_s, shr_r,
          sem_a, sem_b, sem_ca, sem_cb):
        cid = lax.axis_index("c")
        sid = lax.axis_index("s")
        w = cid * _NS + sid

        def zbody(i, _):
            zbuf[i] = jnp.zeros((16,), jnp.float32)
            return 0
        lax.fori_loop(0, 625, zbody, 0)

        pltpu.sync_copy(zbuf, shr_s.at[pl.ds(sid * 625, 625)])
        pltpu.sync_copy(zbuf, shr_r.at[pl.ds(sid * 625, 625)])

        plsc.subcore_barrier()

        pltpu.sync_copy(sidx_hbm.at[pl.ds(w * _RPW, _RPW)], idxs_v)
        pltpu.sync_copy(ridx_hbm.at[pl.ds(w * _RPW, _RPW)], idxr_v)
        bufs, sems = (buf_a, buf_b), (sem_a, sem_b)
        csems = (sem_ca, sem_cb)

        def load_fire(g):
            p = g % 2
            pltpu.async_copy(ne_hbm.at[pl.ds(w * _RPW + g * _KG, _KG)],
                             bufs[p], sems[p])

        def load_drain(g):
            p = g % 2
            pltpu.make_async_copy(
                ne_hbm.at[pl.ds(w * _RPW + g * _KG, _KG)],
                bufs[p], sems[p]).wait()

        def sc_fire(g):
            p = g % 2
            for j in range(_KG):
                pltpu.async_copy(bufs[p].at[j],
                                 shr_s.at[idxs_v.at[g * _KG + j]], csems[p],
                                 add=True)
                pltpu.async_copy(bufs[p].at[j],
                                 shr_r.at[idxr_v.at[g * _KG + j]], csems[p],
                                 add=True)

        def sc_drain(g):
            p = g % 2
            for j in range(_KG):
                pltpu.make_async_copy(
                    bufs[p].at[j], shr_s.at[idxs_v.at[g * _KG + j]],
                    csems[p]).wait()
                pltpu.make_async_copy(
                    bufs[p].at[j], shr_r.at[idxr_v.at[g * _KG + j]],
                    csems[p]).wait()

        load_fire(0)
        for g in range(_NG):
            if g + 1 < _NG:
                if g >= 1:
                    sc_drain(g - 1)
                load_fire(g + 1)
            load_drain(g)
            sc_fire(g)
        sc_drain(_NG - 2)
        sc_drain(_NG - 1)

        plsc.subcore_barrier()

        pltpu.sync_copy(shr_s.at[pl.ds(sid * 625, 625)],
                        sagg_hbm.at[cid, pl.ds(sid * 625, 625), :])
        pltpu.sync_copy(shr_r.at[pl.ds(sid * 625, 625)],
                        ragg_hbm.at[cid, pl.ds(sid * 625, 625), :])

    return k(ne, sidx, ridx)


def _tc_proj(nodes, ws, wr):
    bn = 1000

    def body(n_ref, ws_ref, wr_ref, ps_ref, pr_ref):
        x = n_ref[...]
        ps_ref[...] = jnp.dot(x, ws_ref[...], preferred_element_type=jnp.float32)
        pr_ref[...] = jnp.dot(x, wr_ref[...], preferred_element_type=jnp.float32)

    d = nodes.shape[1]
    return pl.pallas_call(
        body,
        grid=(_N // bn,),
        in_specs=[
            pl.BlockSpec((bn, d), lambda i: (i, 0)),
            pl.BlockSpec((d, 16), lambda i: (0, 0)),
            pl.BlockSpec((d, 16), lambda i: (0, 0)),
        ],
        out_specs=[
            pl.BlockSpec((bn, 16), lambda i: (i, 0)),
            pl.BlockSpec((bn, 16), lambda i: (i, 0)),
        ],
        out_shape=[
            jax.ShapeDtypeStruct((_N, 16), jnp.float32),
            jax.ShapeDtypeStruct((_N, 16), jnp.float32),
        ],
    )(nodes, ws, wr)


def _tc_edge(edges_p, pre, bd1, bd2, c1t, b2t):
    br = 2048

    def body(e_ref, p_ref, bd1_ref, bd2_ref, c1_ref, b2_ref,
             ne_ref, agg_ref):
        i = pl.program_id(0)
        x = jnp.dot(e_ref[...].astype(jnp.bfloat16),
                    bd1_ref[...].astype(jnp.bfloat16),
                    preferred_element_type=jnp.float32)
        x = x + p_ref[...] + c1_ref[...]
        x = x * jax.nn.sigmoid(x)
        y = jnp.dot(x.astype(jnp.bfloat16),
                    bd2_ref[...].astype(jnp.bfloat16),
                    preferred_element_type=jnp.float32)
        y = y + b2_ref[...]
        rows = i * br + lax.broadcasted_iota(jnp.int32, (br, 128), 0)
        y = jnp.where(rows < _E8, y, 0.0)
        ne_ref[...] = y

        @pl.when(i == 0)
        def _():
            agg_ref[...] = jnp.zeros_like(agg_ref)

        agg_ref[...] += jnp.sum(y, axis=0, keepdims=True)

    return pl.pallas_call(
        body,
        grid=(_EPK // br,),
        in_specs=[
            pl.BlockSpec((br, 128), lambda i: (i, 0)),
            pl.BlockSpec((br, 128), lambda i: (i, 0)),
            pl.BlockSpec((128, 128), lambda i: (0, 0)),
            pl.BlockSpec((128, 128), lambda i: (0, 0)),
            pl.BlockSpec((1, 128), lambda i: (0, 0)),
            pl.BlockSpec((1, 128), lambda i: (0, 0)),
        ],
        out_specs=[
            pl.BlockSpec((br, 128), lambda i: (i, 0)),
            pl.BlockSpec((1, 128), lambda i: (0, 0)),
        ],
        out_shape=[
            jax.ShapeDtypeStruct((_EPK, 128), jnp.float32),
            jax.ShapeDtypeStruct((1, 128), jnp.float32),
        ],
    )(edges_p, pre, bd1, bd2, c1t, b2t)


def _tc_node(nodes, sagg2, ragg2, wn_n, wn_s, wn_r, cn, wn2, bn2, wsn, wrn):
    bn = 2000
    d = nodes.shape[1]

    def body(n_ref, s_ref, r_ref, wnn_ref, wns_ref, wnr_ref, cn_ref,
             wn2_ref, bn2_ref, wsn_ref, wrn_ref,
             nn_ref, ps_ref, pr_ref, agg_ref):
        i = pl.program_id(0)
        sa = s_ref[0] + s_ref[1]
        ra = r_ref[0] + r_ref[1]
        h = (jnp.dot(n_ref[...], wnn_ref[...], preferred_element_type=jnp.float32)
             + jnp.dot(sa, wns_ref[...], preferred_element_type=jnp.float32)
             + jnp.dot(ra, wnr_ref[...], preferred_element_type=jnp.float32)
             + cn_ref[...])
        h = h * jax.nn.sigmoid(h)
        nn = jnp.dot(h, wn2_ref[...], preferred_element_type=jnp.float32)
        nn = nn + bn2_ref[...]
        nn_ref[...] = nn
        ps_ref[...] = jnp.dot(nn, wsn_ref[...], preferred_element_type=jnp.float32)
        pr_ref[...] = jnp.dot(nn, wrn_ref[...], preferred_element_type=jnp.float32)

        @pl.when(i == 0)
        def _():
            agg_ref[...] = jnp.zeros_like(agg_ref)

        agg_ref[...] += jnp.sum(nn, axis=0, keepdims=True)

    return pl.pallas_call(
        body,
        grid=(_N // bn,),
        in_specs=[
            pl.BlockSpec((bn, d), lambda i: (i, 0)),
            pl.BlockSpec((_NC, bn, 16), lambda i: (0, i, 0)),
            pl.BlockSpec((_NC, bn, 16), lambda i: (0, i, 0)),
            pl.BlockSpec((d, 16), lambda i: (0, 0)),
            pl.BlockSpec((16, 16), lambda i: (0, 0)),
            pl.BlockSpec((16, 16), lambda i: (0, 0)),
            pl.BlockSpec((1, 16), lambda i: (0, 0)),
            pl.BlockSpec((16, 16), lambda i: (0, 0)),
            pl.BlockSpec((1, 16), lambda i: (0, 0)),
            pl.BlockSpec((16, 16), lambda i: (0, 0)),
            pl.BlockSpec((16, 16), lambda i: (0, 0)),
        ],
        out_specs=[
            pl.BlockSpec((bn, 16), lambda i: (i, 0)),
            pl.BlockSpec((bn, 16), lambda i: (i, 0)),
            pl.BlockSpec((bn, 16), lambda i: (i, 0)),
            pl.BlockSpec((1, 16), lambda i: (0, 0)),
        ],
        out_shape=[
            jax.ShapeDtypeStruct((_N, 16), jnp.float32),
            jax.ShapeDtypeStruct((_N, 16), jnp.float32),
            jax.ShapeDtypeStruct((_N, 16), jnp.float32),
            jax.ShapeDtypeStruct((1, 16), jnp.float32),
        ],
    )(nodes, sagg2, ragg2, wn_n, wn_s, wn_r, cn, wn2, bn2, wsn, wrn)


def _kron8(w):
    return jnp.kron(jnp.eye(8, dtype=jnp.float32), w)


def _tile8(v):
    return jnp.tile(v.reshape(1, 16), (1, 8))


def kernel(nodes, edges, senders, receivers, params):
    blocks = ([params['encoder']] + [params['processor']] * 4
              + [params['decoder']])

    pad = _EP - _E
    sidx = jnp.concatenate(
        [senders, jnp.zeros((pad,), jnp.int32)]).reshape(_NRI, _CH)
    ridx = jnp.concatenate(
        [receivers, jnp.zeros((pad,), jnp.int32)]).reshape(_NRI, _CH)
    edges_p = jnp.concatenate(
        [edges.reshape(_E8, 128),
         jnp.zeros((_EPK - _E8, 128), jnp.float32)])

    g = jnp.zeros((1, 1), jnp.float32)
    nodes_cur = nodes
    zero16 = jnp.zeros((16, 16), jnp.float32)

    def _esplit(b, d_n, d_g):
        (w1, b1), (w2, b2) = b['edge']
        return (w1[:16], w1[16:16 + d_n], w1[16 + d_n:16 + 2 * d_n],
                w1[-d_g:], b1, w2, b2)

    we1_e0, ws0, wr0, _, _, _, _ = _esplit(blocks[0], 128, 1)
    ps, pr = _tc_proj(nodes, ws0, wr0)

    for bi, blk in enumerate(blocks):
        d_n = 128 if bi == 0 else 16
        d_g = g.shape[1]
        we1_e, _, _, we1_g, be1, we2, be2 = _esplit(blk, d_n, d_g)
        c1 = g @ we1_g + be1.reshape(1, 16)

        pre = _sc_gather(ps, pr, sidx, ridx)
        new_edges, eagg_p = _tc_edge(
            edges_p, pre.reshape(_EPK, 128),
            _kron8(we1_e), _kron8(we2), _tile8(c1), _tile8(be2.reshape(1, 16)))
        edge_agg = eagg_p.reshape(8, 16).sum(axis=0, keepdims=True)

        sagg2, ragg2 = _sc_scatter(new_edges.reshape(_NRI, _CH, 16), sidx, ridx)

        (wn1, bn1), (wn2, bn2) = blk['node']
        wn_n = wn1[:d_n]
        wn_s = wn1[d_n:d_n + 16]
        wn_r = wn1[d_n + 16:d_n + 32]
        wn_g = wn1[-d_g:]
        cn = g @ wn_g + bn1.reshape(1, 16)

        if bi + 1 < len(blocks):
            nblk = blocks[bi + 1]
            wsn = nblk['edge'][0][0][16:32]
            wrn = nblk['edge'][0][0][32:48]
        else:
            wsn = wrn = zero16

        new_nodes, ps, pr, node_agg = _tc_node(
            nodes_cur, sagg2, ragg2, wn_n, wn_s, wn_r, cn,
            wn2, bn2.reshape(1, 16), wsn, wrn)

        (wg1, bg1), (wg2, bg2) = blk['global']
        cat = jnp.concatenate([node_agg, edge_agg, g], axis=1)
        hg = cat @ wg1 + bg1.reshape(1, -1)
        hg = hg * jax.nn.sigmoid(hg)
        g = hg @ wg2 + bg2.reshape(1, -1)

        nodes_cur = new_nodes
        edges_p = new_edges

    return g[:, 0]


# speedup vs baseline: 1.1774x; 1.1774x over previous
import functools

import jax
import jax.numpy as jnp
from jax import lax
from jax.experimental import pallas as pl
from jax.experimental.pallas import tpu as pltpu
from jax.experimental.pallas import tpu_sc as plsc

_NC, _NS = 2, 16
_NW = _NC * _NS
_N = 10000
_E = 320000
_EP = 327680
_CH = 128
_NRI = _EP // _CH
_RPW = _NRI // _NW
_KG = 4
_NG = _RPW // _KG
_OUTR = _KG * _CH * 16 // 128
_EPK = _EP * 16 // 128
_E8 = _E * 16 // 128
_RPT = _EPK // _NW


def _sc_gather(ps_t, pr_t, sidx, ridx):
    npt = _N // _NS
    mesh = plsc.VectorSubcoreMesh(core_axis_name="c", subcore_axis_name="s")

    @functools.partial(
        pl.kernel,
        out_type=jax.ShapeDtypeStruct((_NRI, _CH, 16), jnp.float32),
        mesh=mesh,
        compiler_params=pltpu.CompilerParams(use_tc_tiling_on_sc=False),
        scratch_types=[
            pltpu.VMEM((_RPW, _CH), jnp.int32),
            pltpu.VMEM((_RPW, _CH), jnp.int32),
            pltpu.VMEM((_KG, _CH, 16), jnp.float32),
            pltpu.VMEM((_KG, _CH, 16), jnp.float32),
            pltpu.VMEM_SHARED((_N, 16), jnp.float32),
            pltpu.VMEM_SHARED((_N, 16), jnp.float32),
            pltpu.SemaphoreType.DMA,
            pltpu.SemaphoreType.DMA,
            pltpu.SemaphoreType.DMA,
            pltpu.SemaphoreType.DMA,
            pltpu.SemaphoreType.DMA,
            pltpu.SemaphoreType.DMA,
        ],
    )
    def k(ps_hbm, pr_hbm, sidx_hbm, ridx_hbm, out_hbm,
          idxs_v, idxr_v, buf_a, buf_b, tab_s, tab_r,
          sem_sa, sem_sb, sem_ra, sem_rb, sem_oa, sem_ob):
        sid = lax.axis_index("s")
        w = lax.axis_index("c") * _NS + sid
        pltpu.sync_copy(ps_hbm.at[pl.ds(sid * npt, npt)],
                        tab_s.at[pl.ds(sid * npt, npt)])
        pltpu.sync_copy(pr_hbm.at[pl.ds(sid * npt, npt)],
                        tab_r.at[pl.ds(sid * npt, npt)])
        pltpu.sync_copy(sidx_hbm.at[pl.ds(w * _RPW, _RPW)], idxs_v)
        pltpu.sync_copy(ridx_hbm.at[pl.ds(w * _RPW, _RPW)], idxr_v)
        plsc.subcore_barrier()
        bufs = (buf_a, buf_b)
        ssems, rsems, osems = (sem_sa, sem_sb), (sem_ra, sem_rb), (sem_oa, sem_ob)

        def s_fire(g):
            p = g % 2
            for j in range(_KG):
                pltpu.async_copy(
                    tab_s.at[idxs_v.at[g * _KG + j]], bufs[p].at[j], ssems[p])

        def s_drain(g):
            p = g % 2
            for j in range(_KG):
                pltpu.make_async_copy(
                    tab_s.at[idxs_v.at[g * _KG + j]], bufs[p].at[j],
                    ssems[p]).wait()

        def r_fire(g):
            p = g % 2
            for j in range(_KG):
                pltpu.async_copy(
                    tab_r.at[idxr_v.at[g * _KG + j]], bufs[p].at[j], rsems[p],
                    add=True)

        def r_drain(g):
            p = g % 2
            for j in range(_KG):
                pltpu.make_async_copy(
                    tab_r.at[idxr_v.at[g * _KG + j]], bufs[p].at[j],
                    rsems[p]).wait()

        def o_fire(g):
            p = g % 2
            pltpu.async_copy(
                bufs[p], out_hbm.at[pl.ds(w * _RPW + g * _KG, _KG)], osems[p])

        def o_drain(g):
            p = g % 2
            pltpu.make_async_copy(
                bufs[p], out_hbm.at[pl.ds(w * _RPW + g * _KG, _KG)],
                osems[p]).wait()

        s_fire(0)
        for g in range(_NG):
            s_drain(g)
            r_fire(g)
            if g + 1 < _NG:
                if g >= 1:
                    o_drain(g - 1)
                s_fire(g + 1)
            r_drain(g)
            o_fire(g)
        o_drain(_NG - 2)
        o_drain(_NG - 1)

    return k(ps_t, pr_t, sidx, ridx)


def _sc_scatter(ne, sidx, ridx):
    mesh = plsc.VectorSubcoreMesh(core_axis_name="c", subcore_axis_name="s")

    @functools.partial(
        pl.kernel,
        out_type=(
            jax.ShapeDtypeStruct((_NC, _N, 16), jnp.float32),
            jax.ShapeDtypeStruct((_NC, _N, 16), jnp.float32),
        ),
        mesh=mesh,
        compiler_params=pltpu.CompilerParams(use_tc_tiling_on_sc=False),
        scratch_types=[
            pltpu.VMEM((_RPW, _CH), jnp.int32),
            pltpu.VMEM((_RPW, _CH), jnp.int32),
            pltpu.VMEM((_KG, _CH, 16), jnp.float32),
            pltpu.VMEM((_KG, _CH, 16), jnp.float32),
            pltpu.VMEM((625, 16), jnp.float32),
            pltpu.VMEM_SHARED((_N, 16), jnp.float32),
            pltpu.VMEM_SHARED((_N, 16), jnp.float32),
            pltpu.SemaphoreType.DMA,
            pltpu.SemaphoreType.DMA,
            pltpu.SemaphoreType.DMA,
            pltpu.SemaphoreType.DMA,
        ],
    )
    def k(ne_hbm, sidx_hbm, ridx_hbm, sagg_hbm, ragg_hbm,
          idxs_v, idxr_v, buf_a, buf_b, zbuf, shr_s, shr_r,
          sem_a, sem_b, sem_ca, sem_cb):
        cid = lax.axis_index("c")
        sid = lax.axis_index("s")
        w = cid * _NS + sid

        def zbody(i, _):
            zbuf[i] = jnp.zeros((16,), jnp.float32)
            return 0
        lax.fori_loop(0, 625, zbody, 0)

        pltpu.sync_copy(zbuf, shr_s.at[pl.ds(sid * 625, 625)])
        pltpu.sync_copy(zbuf, shr_r.at[pl.ds(sid * 625, 625)])

        plsc.subcore_barrier()

        pltpu.sync_copy(sidx_hbm.at[pl.ds(w * _RPW, _RPW)], idxs_v)
        pltpu.sync_copy(ridx_hbm.at[pl.ds(w * _RPW, _RPW)], idxr_v)
        bufs, sems = (buf_a, buf_b), (sem_a, sem_b)
        csems = (sem_ca, sem_cb)

        def load_fire(g):
            p = g % 2
            pltpu.async_copy(ne_hbm.at[pl.ds(w * _RPW + g * _KG, _KG)],
                             bufs[p], sems[p])

        def load_drain(g):
            p = g % 2
            pltpu.make_async_copy(
                ne_hbm.at[pl.ds(w * _RPW + g * _KG, _KG)],
                bufs[p], sems[p]).wait()

        def sc_fire(g):
            p = g % 2
            for j in range(_KG):
                pltpu.async_copy(bufs[p].at[j],
                                 shr_s.at[idxs_v.at[g * _KG + j]], csems[p],
                                 add=True)
                pltpu.async_copy(bufs[p].at[j],
                                 shr_r.at[idxr_v.at[g * _KG + j]], csems[p],
                                 add=True)

        def sc_drain(g):
            p = g % 2
            for j in range(_KG):
                pltpu.make_async_copy(
                    bufs[p].at[j], shr_s.at[idxs_v.at[g * _KG + j]],
                    csems[p]).wait()
                pltpu.make_async_copy(
                    bufs[p].at[j], shr_r.at[idxr_v.at[g * _KG + j]],
                    csems[p]).wait()

        load_fire(0)
        for g in range(_NG):
            if g + 1 < _NG:
                if g >= 1:
                    sc_drain(g - 1)
                load_fire(g + 1)
            load_drain(g)
            sc_fire(g)
        sc_drain(_NG - 2)
        sc_drain(_NG - 1)

        plsc.subcore_barrier()

        pltpu.sync_copy(shr_s.at[pl.ds(sid * 625, 625)],
                        sagg_hbm.at[cid, pl.ds(sid * 625, 625), :])
        pltpu.sync_copy(shr_r.at[pl.ds(sid * 625, 625)],
                        ragg_hbm.at[cid, pl.ds(sid * 625, 625), :])

    return k(ne, sidx, ridx)


def _tc_proj(nodes, ws, wr):
    bn = 1000

    def body(n_ref, ws_ref, wr_ref, ps_ref, pr_ref):
        x = n_ref[...]
        ps_ref[...] = jnp.dot(x, ws_ref[...], preferred_element_type=jnp.float32)
        pr_ref[...] = jnp.dot(x, wr_ref[...], preferred_element_type=jnp.float32)

    d = nodes.shape[1]
    return pl.pallas_call(
        body,
        grid=(_N // bn,),
        in_specs=[
            pl.BlockSpec((bn, d), lambda i: (i, 0)),
            pl.BlockSpec((d, 16), lambda i: (0, 0)),
            pl.BlockSpec((d, 16), lambda i: (0, 0)),
        ],
        out_specs=[
            pl.BlockSpec((bn, 16), lambda i: (i, 0)),
            pl.BlockSpec((bn, 16), lambda i: (i, 0)),
        ],
        out_shape=[
            jax.ShapeDtypeStruct((_N, 16), jnp.float32),
            jax.ShapeDtypeStruct((_N, 16), jnp.float32),
        ],
    )(nodes, ws, wr)


def _tc_edge(edges_p, pre, bd1, bd2, c1t, b2t):
    br = 2048

    def body(e_ref, p_ref, bd1_ref, bd2_ref, c1_ref, b2_ref,
             ne_ref, agg_ref):
        i = pl.program_id(0)
        x = jnp.dot(e_ref[...], bd1_ref[...], preferred_element_type=jnp.float32)
        x = x + p_ref[...] + c1_ref[...]
        x = x * jax.nn.sigmoid(x)
        y = jnp.dot(x, bd2_ref[...], preferred_element_type=jnp.float32)
        y = y + b2_ref[...]
        rows = i * br + lax.broadcasted_iota(jnp.int32, (br, 128), 0)
        y = jnp.where(rows < _E8, y, 0.0)
        ne_ref[...] = y

        @pl.when(i == 0)
        def _():
            agg_ref[...] = jnp.zeros_like(agg_ref)

        agg_ref[...] += jnp.sum(y, axis=0, keepdims=True)

    return pl.pallas_call(
        body,
        grid=(_EPK // br,),
        in_specs=[
            pl.BlockSpec((br, 128), lambda i: (i, 0)),
            pl.BlockSpec((br, 128), lambda i: (i, 0)),
            pl.BlockSpec((128, 128), lambda i: (0, 0)),
            pl.BlockSpec((128, 128), lambda i: (0, 0)),
            pl.BlockSpec((1, 128), lambda i: (0, 0)),
            pl.BlockSpec((1, 128), lambda i: (0, 0)),
        ],
        out_specs=[
            pl.BlockSpec((br, 128), lambda i: (i, 0)),
            pl.BlockSpec((1, 128), lambda i: (0, 0)),
        ],
        out_shape=[
            jax.ShapeDtypeStruct((_EPK, 128), jnp.float32),
            jax.ShapeDtypeStruct((1, 128), jnp.float32),
        ],
    )(edges_p, pre, bd1, bd2, c1t, b2t)


def _tc_node(nodes, sagg2, ragg2, wn_n, wn_s, wn_r, cn, wn2, bn2, wsn, wrn):
    bn = 2000
    d = nodes.shape[1]

    def body(n_ref, s_ref, r_ref, wnn_ref, wns_ref, wnr_ref, cn_ref,
             wn2_ref, bn2_ref, wsn_ref, wrn_ref,
             nn_ref, ps_ref, pr_ref, agg_ref):
        i = pl.program_id(0)
        sa = s_ref[0] + s_ref[1]
        ra = r_ref[0] + r_ref[1]
        h = (jnp.dot(n_ref[...], wnn_ref[...], preferred_element_type=jnp.float32)
             + jnp.dot(sa, wns_ref[...], preferred_element_type=jnp.float32)
             + jnp.dot(ra, wnr_ref[...], preferred_element_type=jnp.float32)
             + cn_ref[...])
        h = h * jax.nn.sigmoid(h)
        nn = jnp.dot(h, wn2_ref[...], preferred_element_type=jnp.float32)
        nn = nn + bn2_ref[...]
        nn_ref[...] = nn
        ps_ref[...] = jnp.dot(nn, wsn_ref[...], preferred_element_type=jnp.float32)
        pr_ref[...] = jnp.dot(nn, wrn_ref[...], preferred_element_type=jnp.float32)

        @pl.when(i == 0)
        def _():
            agg_ref[...] = jnp.zeros_like(agg_ref)

        agg_ref[...] += jnp.sum(nn, axis=0, keepdims=True)

    return pl.pallas_call(
        body,
        grid=(_N // bn,),
        in_specs=[
            pl.BlockSpec((bn, d), lambda i: (i, 0)),
            pl.BlockSpec((_NC, bn, 16), lambda i: (0, i, 0)),
            pl.BlockSpec((_NC, bn, 16), lambda i: (0, i, 0)),
            pl.BlockSpec((d, 16), lambda i: (0, 0)),
            pl.BlockSpec((16, 16), lambda i: (0, 0)),
            pl.BlockSpec((16, 16), lambda i: (0, 0)),
            pl.BlockSpec((1, 16), lambda i: (0, 0)),
            pl.BlockSpec((16, 16), lambda i: (0, 0)),
            pl.BlockSpec((1, 16), lambda i: (0, 0)),
            pl.BlockSpec((16, 16), lambda i: (0, 0)),
            pl.BlockSpec((16, 16), lambda i: (0, 0)),
        ],
        out_specs=[
            pl.BlockSpec((bn, 16), lambda i: (i, 0)),
            pl.BlockSpec((bn, 16), lambda i: (i, 0)),
            pl.BlockSpec((bn, 16), lambda i: (i, 0)),
            pl.BlockSpec((1, 16), lambda i: (0, 0)),
        ],
        out_shape=[
            jax.ShapeDtypeStruct((_N, 16), jnp.float32),
            jax.ShapeDtypeStruct((_N, 16), jnp.float32),
            jax.ShapeDtypeStruct((_N, 16), jnp.float32),
            jax.ShapeDtypeStruct((1, 16), jnp.float32),
        ],
    )(nodes, sagg2, ragg2, wn_n, wn_s, wn_r, cn, wn2, bn2, wsn, wrn)


def _tc_node_packed(nodes_pk, sagg_pk, ragg_pk, bdn, bds, bdr, cnt,
                    bd2, bn2t, bdsn, bdrn):
    npk = _N // 8

    def body(n_ref, s_ref, r_ref, bdn_ref, bds_ref, bdr_ref, cn_ref,
             bd2_ref, bn2_ref, bdsn_ref, bdrn_ref,
             nn_ref, ps_ref, pr_ref, agg_ref):
        sa = s_ref[0] + s_ref[1]
        ra = r_ref[0] + r_ref[1]
        h = (jnp.dot(n_ref[...], bdn_ref[...], preferred_element_type=jnp.float32)
             + jnp.dot(sa, bds_ref[...], preferred_element_type=jnp.float32)
             + jnp.dot(ra, bdr_ref[...], preferred_element_type=jnp.float32)
             + cn_ref[...])
        h = h * jax.nn.sigmoid(h)
        nn = jnp.dot(h, bd2_ref[...], preferred_element_type=jnp.float32)
        nn = nn + bn2_ref[...]
        nn_ref[...] = nn
        ps_ref[...] = jnp.dot(nn, bdsn_ref[...], preferred_element_type=jnp.float32)
        pr_ref[...] = jnp.dot(nn, bdrn_ref[...], preferred_element_type=jnp.float32)
        agg_ref[...] = jnp.sum(nn, axis=0, keepdims=True)

    full = lambda s: pl.BlockSpec(s, lambda: tuple(0 for _ in s))
    return pl.pallas_call(
        body,
        in_specs=[
            full((npk, 128)),
            full((_NC, npk, 128)),
            full((_NC, npk, 128)),
            full((128, 128)), full((128, 128)), full((128, 128)),
            full((1, 128)),
            full((128, 128)), full((1, 128)),
            full((128, 128)), full((128, 128)),
        ],
        out_specs=[
            full((npk, 128)), full((npk, 128)), full((npk, 128)),
            full((1, 128)),
        ],
        out_shape=[
            jax.ShapeDtypeStruct((npk, 128), jnp.float32),
            jax.ShapeDtypeStruct((npk, 128), jnp.float32),
            jax.ShapeDtypeStruct((npk, 128), jnp.float32),
            jax.ShapeDtypeStruct((1, 128), jnp.float32),
        ],
    )(nodes_pk, sagg_pk, ragg_pk, bdn, bds, bdr, cnt, bd2, bn2t, bdsn, bdrn)


def _kron8(w):
    return jnp.kron(jnp.eye(8, dtype=jnp.float32), w)


def _tile8(v):
    return jnp.tile(v.reshape(1, 16), (1, 8))


def kernel(nodes, edges, senders, receivers, params):
    blocks = ([params['encoder']] + [params['processor']] * 4
              + [params['decoder']])

    pad = _EP - _E
    sidx = jnp.concatenate(
        [senders, jnp.zeros((pad,), jnp.int32)]).reshape(_NRI, _CH)
    ridx = jnp.concatenate(
        [receivers, jnp.zeros((pad,), jnp.int32)]).reshape(_NRI, _CH)
    edges_p = jnp.concatenate(
        [edges.reshape(_E8, 128),
         jnp.zeros((_EPK - _E8, 128), jnp.float32)])

    g = jnp.zeros((1, 1), jnp.float32)
    nodes_cur = nodes
    zero16 = jnp.zeros((16, 16), jnp.float32)

    def _esplit(b, d_n, d_g):
        (w1, b1), (w2, b2) = b['edge']
        return (w1[:16], w1[16:16 + d_n], w1[16 + d_n:16 + 2 * d_n],
                w1[-d_g:], b1, w2, b2)

    we1_e0, ws0, wr0, _, _, _, _ = _esplit(blocks[0], 128, 1)
    ps, pr = _tc_proj(nodes, ws0, wr0)

    for bi, blk in enumerate(blocks):
        d_n = 128 if bi == 0 else 16
        d_g = g.shape[1]
        we1_e, _, _, we1_g, be1, we2, be2 = _esplit(blk, d_n, d_g)
        c1 = g @ we1_g + be1.reshape(1, 16)

        pre = _sc_gather(ps, pr, sidx, ridx)
        new_edges, eagg_p = _tc_edge(
            edges_p, pre.reshape(_EPK, 128),
            _kron8(we1_e), _kron8(we2), _tile8(c1), _tile8(be2.reshape(1, 16)))
        edge_agg = eagg_p.reshape(8, 16).sum(axis=0, keepdims=True)

        sagg2, ragg2 = _sc_scatter(new_edges.reshape(_NRI, _CH, 16), sidx, ridx)

        (wn1, bn1), (wn2, bn2) = blk['node']
        wn_n = wn1[:d_n]
        wn_s = wn1[d_n:d_n + 16]
        wn_r = wn1[d_n + 16:d_n + 32]
        wn_g = wn1[-d_g:]
        cn = g @ wn_g + bn1.reshape(1, 16)

        if bi + 1 < len(blocks):
            nblk = blocks[bi + 1]
            wsn = nblk['edge'][0][0][16:32]
            wrn = nblk['edge'][0][0][32:48]
        else:
            wsn = wrn = zero16

        if bi == 0:
            new_nodes, ps, pr, node_agg = _tc_node(
                nodes_cur, sagg2, ragg2, wn_n, wn_s, wn_r, cn,
                wn2, bn2.reshape(1, 16), wsn, wrn)
            nodes_cur = new_nodes.reshape(_N // 8, 128)
        else:
            nodes_cur, ps_pk, pr_pk, nagg_p = _tc_node_packed(
                nodes_cur, sagg2.reshape(_NC, _N // 8, 128),
                ragg2.reshape(_NC, _N // 8, 128),
                _kron8(wn_n), _kron8(wn_s), _kron8(wn_r), _tile8(cn),
                _kron8(wn2), _tile8(bn2.reshape(1, 16)),
                _kron8(wsn), _kron8(wrn))
            ps = ps_pk.reshape(_N, 16)
            pr = pr_pk.reshape(_N, 16)
            node_agg = nagg_p.reshape(8, 16).sum(axis=0, keepdims=True)

        (wg1, bg1), (wg2, bg2) = blk['global']
        cat = jnp.concatenate([node_agg, edge_agg, g], axis=1)
        hg = cat @ wg1 + bg1.reshape(1, -1)
        hg = hg * jax.nn.sigmoid(hg)
        g = hg @ wg2 + bg2.reshape(1, -1)

        edges_p = new_edges

    return g[:, 0]


# speedup vs baseline: 1.2266x; 1.0418x over previous
import functools

import jax
import jax.numpy as jnp
from jax import lax
from jax.experimental import pallas as pl
from jax.experimental.pallas import tpu as pltpu
from jax.experimental.pallas import tpu_sc as plsc

_NC, _NS = 2, 16
_NW = _NC * _NS
_N = 10000
_E = 320000
_EP = 327680
_CH = 128
_NRI = _EP // _CH
_RPW = _NRI // _NW
_KG = 8
_NG = _RPW // _KG
_OUTR = _KG * _CH * 16 // 128
_EPK = _EP * 16 // 128
_E8 = _E * 16 // 128
_RPT = _EPK // _NW


def _sc_gather(ps_t, pr_t, sidx, ridx):
    npt = _N // _NS
    mesh = plsc.VectorSubcoreMesh(core_axis_name="c", subcore_axis_name="s")

    @functools.partial(
        pl.kernel,
        out_type=jax.ShapeDtypeStruct((_NRI, _CH, 16), jnp.float32),
        mesh=mesh,
        compiler_params=pltpu.CompilerParams(use_tc_tiling_on_sc=False),
        scratch_types=[
            pltpu.VMEM((_RPW, _CH), jnp.int32),
            pltpu.VMEM((_RPW, _CH), jnp.int32),
            pltpu.VMEM((_KG, _CH, 16), jnp.float32),
            pltpu.VMEM((_KG, _CH, 16), jnp.float32),
            pltpu.VMEM_SHARED((_N, 16), jnp.float32),
            pltpu.VMEM_SHARED((_N, 16), jnp.float32),
            pltpu.SemaphoreType.DMA,
            pltpu.SemaphoreType.DMA,
            pltpu.SemaphoreType.DMA,
            pltpu.SemaphoreType.DMA,
            pltpu.SemaphoreType.DMA,
            pltpu.SemaphoreType.DMA,
        ],
    )
    def k(ps_hbm, pr_hbm, sidx_hbm, ridx_hbm, out_hbm,
          idxs_v, idxr_v, buf_a, buf_b, tab_s, tab_r,
          sem_sa, sem_sb, sem_ra, sem_rb, sem_oa, sem_ob):
        sid = lax.axis_index("s")
        w = lax.axis_index("c") * _NS + sid
        pltpu.sync_copy(ps_hbm.at[pl.ds(sid * npt, npt)],
                        tab_s.at[pl.ds(sid * npt, npt)])
        pltpu.sync_copy(pr_hbm.at[pl.ds(sid * npt, npt)],
                        tab_r.at[pl.ds(sid * npt, npt)])
        pltpu.sync_copy(sidx_hbm.at[pl.ds(w * _RPW, _RPW)], idxs_v)
        pltpu.sync_copy(ridx_hbm.at[pl.ds(w * _RPW, _RPW)], idxr_v)
        plsc.subcore_barrier()
        bufs = (buf_a, buf_b)
        ssems, rsems, osems = (sem_sa, sem_sb), (sem_ra, sem_rb), (sem_oa, sem_ob)

        def s_fire(g):
            p = g % 2
            for j in range(_KG):
                pltpu.async_copy(
                    tab_s.at[idxs_v.at[g * _KG + j]], bufs[p].at[j], ssems[p])

        def s_drain(g):
            p = g % 2
            for j in range(_KG):
                pltpu.make_async_copy(
                    tab_s.at[idxs_v.at[g * _KG + j]], bufs[p].at[j],
                    ssems[p]).wait()

        def r_fire(g):
            p = g % 2
            for j in range(_KG):
                pltpu.async_copy(
                    tab_r.at[idxr_v.at[g * _KG + j]], bufs[p].at[j], rsems[p],
                    add=True)

        def r_drain(g):
            p = g % 2
            for j in range(_KG):
                pltpu.make_async_copy(
                    tab_r.at[idxr_v.at[g * _KG + j]], bufs[p].at[j],
                    rsems[p]).wait()

        def o_fire(g):
            p = g % 2
            pltpu.async_copy(
                bufs[p], out_hbm.at[pl.ds(w * _RPW + g * _KG, _KG)], osems[p])

        def o_drain(g):
            p = g % 2
            pltpu.make_async_copy(
                bufs[p], out_hbm.at[pl.ds(w * _RPW + g * _KG, _KG)],
                osems[p]).wait()

        s_fire(0)
        for g in range(_NG):
            s_drain(g)
            r_fire(g)
            if g + 1 < _NG:
                if g >= 1:
                    o_drain(g - 1)
                s_fire(g + 1)
            r_drain(g)
            o_fire(g)
        o_drain(_NG - 2)
        o_drain(_NG - 1)

    return k(ps_t, pr_t, sidx, ridx)


def _sc_scatter(ne, sidx, ridx):
    mesh = plsc.VectorSubcoreMesh(core_axis_name="c", subcore_axis_name="s")

    @functools.partial(
        pl.kernel,
        out_type=(
            jax.ShapeDtypeStruct((_NC, _N, 16), jnp.float32),
            jax.ShapeDtypeStruct((_NC, _N, 16), jnp.float32),
        ),
        mesh=mesh,
        compiler_params=pltpu.CompilerParams(use_tc_tiling_on_sc=False),
        scratch_types=[
            pltpu.VMEM((_RPW, _CH), jnp.int32),
            pltpu.VMEM((_RPW, _CH), jnp.int32),
            pltpu.VMEM((_KG, _CH, 16), jnp.float32),
            pltpu.VMEM((_KG, _CH, 16), jnp.float32),
            pltpu.VMEM((625, 16), jnp.float32),
            pltpu.VMEM_SHARED((_N, 16), jnp.float32),
            pltpu.VMEM_SHARED((_N, 16), jnp.float32),
            pltpu.SemaphoreType.DMA,
            pltpu.SemaphoreType.DMA,
            pltpu.SemaphoreType.DMA,
            pltpu.SemaphoreType.DMA,
        ],
    )
    def k(ne_hbm, sidx_hbm, ridx_hbm, sagg_hbm, ragg_hbm,
          idxs_v, idxr_v, buf_a, buf_b, zbuf, shr_s, shr_r,
          sem_a, sem_b, sem_ca, sem_cb):
        cid = lax.axis_index("c")
        sid = lax.axis_index("s")
        w = cid * _NS + sid

        def zbody(i, _):
            zbuf[i] = jnp.zeros((16,), jnp.float32)
            return 0
        lax.fori_loop(0, 625, zbody, 0)

        pltpu.sync_copy(zbuf, shr_s.at[pl.ds(sid * 625, 625)])
        pltpu.sync_copy(zbuf, shr_r.at[pl.ds(sid * 625, 625)])

        plsc.subcore_barrier()

        pltpu.sync_copy(sidx_hbm.at[pl.ds(w * _RPW, _RPW)], idxs_v)
        pltpu.sync_copy(ridx_hbm.at[pl.ds(w * _RPW, _RPW)], idxr_v)
        bufs, sems = (buf_a, buf_b), (sem_a, sem_b)
        csems = (sem_ca, sem_cb)

        def load_fire(g):
            p = g % 2
            pltpu.async_copy(ne_hbm.at[pl.ds(w * _RPW + g * _KG, _KG)],
                             bufs[p], sems[p])

        def load_drain(g):
            p = g % 2
            pltpu.make_async_copy(
                ne_hbm.at[pl.ds(w * _RPW + g * _KG, _KG)],
                bufs[p], sems[p]).wait()

        def sc_fire(g):
            p = g % 2
            for j in range(_KG):
                pltpu.async_copy(bufs[p].at[j],
                                 shr_s.at[idxs_v.at[g * _KG + j]], csems[p],
                                 add=True)
                pltpu.async_copy(bufs[p].at[j],
                                 shr_r.at[idxr_v.at[g * _KG + j]], csems[p],
                                 add=True)

        def sc_drain(g):
            p = g % 2
            for j in range(_KG):
                pltpu.make_async_copy(
                    bufs[p].at[j], shr_s.at[idxs_v.at[g * _KG + j]],
                    csems[p]).wait()
                pltpu.make_async_copy(
                    bufs[p].at[j], shr_r.at[idxr_v.at[g * _KG + j]],
                    csems[p]).wait()

        load_fire(0)
        for g in range(_NG):
            if g + 1 < _NG:
                if g >= 1:
                    sc_drain(g - 1)
                load_fire(g + 1)
            load_drain(g)
            sc_fire(g)
        sc_drain(_NG - 2)
        sc_drain(_NG - 1)

        plsc.subcore_barrier()

        pltpu.sync_copy(shr_s.at[pl.ds(sid * 625, 625)],
                        sagg_hbm.at[cid, pl.ds(sid * 625, 625), :])
        pltpu.sync_copy(shr_r.at[pl.ds(sid * 625, 625)],
                        ragg_hbm.at[cid, pl.ds(sid * 625, 625), :])

    return k(ne, sidx, ridx)


def _tc_proj(nodes, ws, wr):
    bn = 1000

    def body(n_ref, ws_ref, wr_ref, ps_ref, pr_ref):
        x = n_ref[...]
        ps_ref[...] = jnp.dot(x, ws_ref[...], preferred_element_type=jnp.float32)
        pr_ref[...] = jnp.dot(x, wr_ref[...], preferred_element_type=jnp.float32)

    d = nodes.shape[1]
    return pl.pallas_call(
        body,
        grid=(_N // bn,),
        in_specs=[
            pl.BlockSpec((bn, d), lambda i: (i, 0)),
            pl.BlockSpec((d, 16), lambda i: (0, 0)),
            pl.BlockSpec((d, 16), lambda i: (0, 0)),
        ],
        out_specs=[
            pl.BlockSpec((bn, 16), lambda i: (i, 0)),
            pl.BlockSpec((bn, 16), lambda i: (i, 0)),
        ],
        out_shape=[
            jax.ShapeDtypeStruct((_N, 16), jnp.float32),
            jax.ShapeDtypeStruct((_N, 16), jnp.float32),
        ],
    )(nodes, ws, wr)


def _tc_edge(edges_p, pre, bd1, bd2, c1t, b2t):
    br = 4096

    def body(e_ref, p_ref, bd1_ref, bd2_ref, c1_ref, b2_ref,
             ne_ref, agg_ref):
        i = pl.program_id(0)
        x = jnp.dot(e_ref[...], bd1_ref[...], preferred_element_type=jnp.float32)
        x = x + p_ref[...] + c1_ref[...]
        x = x * jax.nn.sigmoid(x)
        y = jnp.dot(x, bd2_ref[...], preferred_element_type=jnp.float32)
        y = y + b2_ref[...]
        rows = i * br + lax.broadcasted_iota(jnp.int32, (br, 128), 0)
        y = jnp.where(rows < _E8, y, 0.0)
        ne_ref[...] = y

        @pl.when(i == 0)
        def _():
            agg_ref[...] = jnp.zeros_like(agg_ref)

        agg_ref[...] += jnp.sum(y, axis=0, keepdims=True)

    return pl.pallas_call(
        body,
        grid=(_EPK // br,),
        in_specs=[
            pl.BlockSpec((br, 128), lambda i: (i, 0)),
            pl.BlockSpec((br, 128), lambda i: (i, 0)),
            pl.BlockSpec((128, 128), lambda i: (0, 0)),
            pl.BlockSpec((128, 128), lambda i: (0, 0)),
            pl.BlockSpec((1, 128), lambda i: (0, 0)),
            pl.BlockSpec((1, 128), lambda i: (0, 0)),
        ],
        out_specs=[
            pl.BlockSpec((br, 128), lambda i: (i, 0)),
            pl.BlockSpec((1, 128), lambda i: (0, 0)),
        ],
        out_shape=[
            jax.ShapeDtypeStruct((_EPK, 128), jnp.float32),
            jax.ShapeDtypeStruct((1, 128), jnp.float32),
        ],
    )(edges_p, pre, bd1, bd2, c1t, b2t)


def _tc_node(nodes, sagg2, ragg2, wn_n, wn_s, wn_r, cn, wn2, bn2, wsn, wrn):
    bn = 2000
    d = nodes.shape[1]

    def body(n_ref, s_ref, r_ref, wnn_ref, wns_ref, wnr_ref, cn_ref,
             wn2_ref, bn2_ref, wsn_ref, wrn_ref,
             nn_ref, ps_ref, pr_ref, agg_ref):
        i = pl.program_id(0)
        sa = s_ref[0] + s_ref[1]
        ra = r_ref[0] + r_ref[1]
        h = (jnp.dot(n_ref[...], wnn_ref[...], preferred_element_type=jnp.float32)
             + jnp.dot(sa, wns_ref[...], preferred_element_type=jnp.float32)
             + jnp.dot(ra, wnr_ref[...], preferred_element_type=jnp.float32)
             + cn_ref[...])
        h = h * jax.nn.sigmoid(h)
        nn = jnp.dot(h, wn2_ref[...], preferred_element_type=jnp.float32)
        nn = nn + bn2_ref[...]
        nn_ref[...] = nn
        ps_ref[...] = jnp.dot(nn, wsn_ref[...], preferred_element_type=jnp.float32)
        pr_ref[...] = jnp.dot(nn, wrn_ref[...], preferred_element_type=jnp.float32)

        @pl.when(i == 0)
        def _():
            agg_ref[...] = jnp.zeros_like(agg_ref)

        agg_ref[...] += jnp.sum(nn, axis=0, keepdims=True)

    return pl.pallas_call(
        body,
        grid=(_N // bn,),
        in_specs=[
            pl.BlockSpec((bn, d), lambda i: (i, 0)),
            pl.BlockSpec((_NC, bn, 16), lambda i: (0, i, 0)),
            pl.BlockSpec((_NC, bn, 16), lambda i: (0, i, 0)),
            pl.BlockSpec((d, 16), lambda i: (0, 0)),
            pl.BlockSpec((16, 16), lambda i: (0, 0)),
            pl.BlockSpec((16, 16), lambda i: (0, 0)),
            pl.BlockSpec((1, 16), lambda i: (0, 0)),
            pl.BlockSpec((16, 16), lambda i: (0, 0)),
            pl.BlockSpec((1, 16), lambda i: (0, 0)),
            pl.BlockSpec((16, 16), lambda i: (0, 0)),
            pl.BlockSpec((16, 16), lambda i: (0, 0)),
        ],
        out_specs=[
            pl.BlockSpec((bn, 16), lambda i: (i, 0)),
            pl.BlockSpec((bn, 16), lambda i: (i, 0)),
            pl.BlockSpec((bn, 16), lambda i: (i, 0)),
            pl.BlockSpec((1, 16), lambda i: (0, 0)),
        ],
        out_shape=[
            jax.ShapeDtypeStruct((_N, 16), jnp.float32),
            jax.ShapeDtypeStruct((_N, 16), jnp.float32),
            jax.ShapeDtypeStruct((_N, 16), jnp.float32),
            jax.ShapeDtypeStruct((1, 16), jnp.float32),
        ],
    )(nodes, sagg2, ragg2, wn_n, wn_s, wn_r, cn, wn2, bn2, wsn, wrn)


def _tc_node_packed(nodes_pk, sagg_pk, ragg_pk, bdn, bds, bdr, cnt,
                    bd2, bn2t, bdsn, bdrn):
    npk = _N // 8

    def body(n_ref, s_ref, r_ref, bdn_ref, bds_ref, bdr_ref, cn_ref,
             bd2_ref, bn2_ref, bdsn_ref, bdrn_ref,
             nn_ref, ps_ref, pr_ref, agg_ref):
        sa = s_ref[0] + s_ref[1]
        ra = r_ref[0] + r_ref[1]
        h = (jnp.dot(n_ref[...], bdn_ref[...], preferred_element_type=jnp.float32)
             + jnp.dot(sa, bds_ref[...], preferred_element_type=jnp.float32)
             + jnp.dot(ra, bdr_ref[...], preferred_element_type=jnp.float32)
             + cn_ref[...])
        h = h * jax.nn.sigmoid(h)
        nn = jnp.dot(h, bd2_ref[...], preferred_element_type=jnp.float32)
        nn = nn + bn2_ref[...]
        nn_ref[...] = nn
        ps_ref[...] = jnp.dot(nn, bdsn_ref[...], preferred_element_type=jnp.float32)
        pr_ref[...] = jnp.dot(nn, bdrn_ref[...], preferred_element_type=jnp.float32)
        agg_ref[...] = jnp.sum(nn, axis=0, keepdims=True)

    full = lambda s: pl.BlockSpec(s, lambda: tuple(0 for _ in s))
    return pl.pallas_call(
        body,
        in_specs=[
            full((npk, 128)),
            full((_NC, npk, 128)),
            full((_NC, npk, 128)),
            full((128, 128)), full((128, 128)), full((128, 128)),
            full((1, 128)),
            full((128, 128)), full((1, 128)),
            full((128, 128)), full((128, 128)),
        ],
        out_specs=[
            full((npk, 128)), full((npk, 128)), full((npk, 128)),
            full((1, 128)),
        ],
        out_shape=[
            jax.ShapeDtypeStruct((npk, 128), jnp.float32),
            jax.ShapeDtypeStruct((npk, 128), jnp.float32),
            jax.ShapeDtypeStruct((npk, 128), jnp.float32),
            jax.ShapeDtypeStruct((1, 128), jnp.float32),
        ],
    )(nodes_pk, sagg_pk, ragg_pk, bdn, bds, bdr, cnt, bd2, bn2t, bdsn, bdrn)


def _kron8(w):
    return jnp.kron(jnp.eye(8, dtype=jnp.float32), w)


def _tile8(v):
    return jnp.tile(v.reshape(1, 16), (1, 8))


def kernel(nodes, edges, senders, receivers, params):
    blocks = ([params['encoder']] + [params['processor']] * 4
              + [params['decoder']])

    pad = _EP - _E
    sidx = jnp.concatenate(
        [senders, jnp.zeros((pad,), jnp.int32)]).reshape(_NRI, _CH)
    ridx = jnp.concatenate(
        [receivers, jnp.zeros((pad,), jnp.int32)]).reshape(_NRI, _CH)
    edges_p = jnp.concatenate(
        [edges.reshape(_E8, 128),
         jnp.zeros((_EPK - _E8, 128), jnp.float32)])

    g = jnp.zeros((1, 1), jnp.float32)
    nodes_cur = nodes
    zero16 = jnp.zeros((16, 16), jnp.float32)

    def _esplit(b, d_n, d_g):
        (w1, b1), (w2, b2) = b['edge']
        return (w1[:16], w1[16:16 + d_n], w1[16 + d_n:16 + 2 * d_n],
                w1[-d_g:], b1, w2, b2)

    we1_e0, ws0, wr0, _, _, _, _ = _esplit(blocks[0], 128, 1)
    ps, pr = _tc_proj(nodes, ws0, wr0)

    for bi, blk in enumerate(blocks):
        d_n = 128 if bi == 0 else 16
        d_g = g.shape[1]
        we1_e, _, _, we1_g, be1, we2, be2 = _esplit(blk, d_n, d_g)
        c1 = g @ we1_g + be1.reshape(1, 16)

        pre = _sc_gather(ps, pr, sidx, ridx)
        new_edges, eagg_p = _tc_edge(
            edges_p, pre.reshape(_EPK, 128),
            _kron8(we1_e), _kron8(we2), _tile8(c1), _tile8(be2.reshape(1, 16)))
        edge_agg = eagg_p.reshape(8, 16).sum(axis=0, keepdims=True)

        sagg2, ragg2 = _sc_scatter(new_edges.reshape(_NRI, _CH, 16), sidx, ridx)

        (wn1, bn1), (wn2, bn2) = blk['node']
        wn_n = wn1[:d_n]
        wn_s = wn1[d_n:d_n + 16]
        wn_r = wn1[d_n + 16:d_n + 32]
        wn_g = wn1[-d_g:]
        cn = g @ wn_g + bn1.reshape(1, 16)

        if bi + 1 < len(blocks):
            nblk = blocks[bi + 1]
            wsn = nblk['edge'][0][0][16:32]
            wrn = nblk['edge'][0][0][32:48]
        else:
            wsn = wrn = zero16

        if bi == 0:
            new_nodes, ps, pr, node_agg = _tc_node(
                nodes_cur, sagg2, ragg2, wn_n, wn_s, wn_r, cn,
                wn2, bn2.reshape(1, 16), wsn, wrn)
            nodes_cur = new_nodes.reshape(_N // 8, 128)
        else:
            nodes_cur, ps_pk, pr_pk, nagg_p = _tc_node_packed(
                nodes_cur, sagg2.reshape(_NC, _N // 8, 128),
                ragg2.reshape(_NC, _N // 8, 128),
                _kron8(wn_n), _kron8(wn_s), _kron8(wn_r), _tile8(cn),
                _kron8(wn2), _tile8(bn2.reshape(1, 16)),
                _kron8(wsn), _kron8(wrn))
            ps = ps_pk.reshape(_N, 16)
            pr = pr_pk.reshape(_N, 16)
            node_agg = nagg_p.reshape(8, 16).sum(axis=0, keepdims=True)

        (wg1, bg1), (wg2, bg2) = blk['global']
        cat = jnp.concatenate([node_agg, edge_agg, g], axis=1)
        hg = cat @ wg1 + bg1.reshape(1, -1)
        hg = hg * jax.nn.sigmoid(hg)
        g = hg @ wg2 + bg2.reshape(1, -1)

        edges_p = new_edges

    return g[:, 0]


# speedup vs baseline: 1.2545x; 1.0227x over previous
import functools

import jax
import jax.numpy as jnp
from jax import lax
from jax.experimental import pallas as pl
from jax.experimental.pallas import tpu as pltpu
from jax.experimental.pallas import tpu_sc as plsc

_NC, _NS = 2, 16
_NW = _NC * _NS
_N = 10000
_E = 320000
_EP = 327680
_CH = 128
_NRI = _EP // _CH
_RPW = _NRI // _NW
_KG = 8
_NG = _RPW // _KG
_OUTR = _KG * _CH * 16 // 128
_EPK = _EP * 16 // 128
_E8 = _E * 16 // 128
_RPT = _EPK // _NW


def _sc_gather(ps_t, pr_t, sidx, ridx):
    npt = _N // _NS
    mesh = plsc.VectorSubcoreMesh(core_axis_name="c", subcore_axis_name="s")

    @functools.partial(
        pl.kernel,
        out_type=jax.ShapeDtypeStruct((_NRI, _CH, 16), jnp.float32),
        mesh=mesh,
        compiler_params=pltpu.CompilerParams(use_tc_tiling_on_sc=False),
        scratch_types=[
            pltpu.VMEM((_RPW, _CH), jnp.int32),
            pltpu.VMEM((_RPW, _CH), jnp.int32),
            pltpu.VMEM((_KG, _CH, 16), jnp.float32),
            pltpu.VMEM((_KG, _CH, 16), jnp.float32),
            pltpu.VMEM_SHARED((_N, 16), jnp.float32),
            pltpu.VMEM_SHARED((_N, 16), jnp.float32),
            pltpu.SemaphoreType.DMA,
            pltpu.SemaphoreType.DMA,
            pltpu.SemaphoreType.DMA,
            pltpu.SemaphoreType.DMA,
            pltpu.SemaphoreType.DMA,
            pltpu.SemaphoreType.DMA,
        ],
    )
    def k(ps_hbm, pr_hbm, sidx_hbm, ridx_hbm, out_hbm,
          idxs_v, idxr_v, buf_a, buf_b, tab_s, tab_r,
          sem_sa, sem_sb, sem_ra, sem_rb, sem_oa, sem_ob):
        sid = lax.axis_index("s")
        w = lax.axis_index("c") * _NS + sid
        pltpu.sync_copy(ps_hbm.at[pl.ds(sid * npt, npt)],
                        tab_s.at[pl.ds(sid * npt, npt)])
        pltpu.sync_copy(pr_hbm.at[pl.ds(sid * npt, npt)],
                        tab_r.at[pl.ds(sid * npt, npt)])
        pltpu.sync_copy(sidx_hbm.at[pl.ds(w * _RPW, _RPW)], idxs_v)
        pltpu.sync_copy(ridx_hbm.at[pl.ds(w * _RPW, _RPW)], idxr_v)
        plsc.subcore_barrier()
        bufs = (buf_a, buf_b)
        ssems, rsems, osems = (sem_sa, sem_sb), (sem_ra, sem_rb), (sem_oa, sem_ob)

        def s_fire(g):
            p = g % 2
            for j in range(_KG):
                pltpu.async_copy(
                    tab_s.at[idxs_v.at[g * _KG + j]], bufs[p].at[j], ssems[p])

        def s_drain(g):
            p = g % 2
            for j in range(_KG):
                pltpu.make_async_copy(
                    tab_s.at[idxs_v.at[g * _KG + j]], bufs[p].at[j],
                    ssems[p]).wait()

        def r_fire(g):
            p = g % 2
            for j in range(_KG):
                pltpu.async_copy(
                    tab_r.at[idxr_v.at[g * _KG + j]], bufs[p].at[j], rsems[p],
                    add=True)

        def r_drain(g):
            p = g % 2
            for j in range(_KG):
                pltpu.make_async_copy(
                    tab_r.at[idxr_v.at[g * _KG + j]], bufs[p].at[j],
                    rsems[p]).wait()

        def o_fire(g):
            p = g % 2
            pltpu.async_copy(
                bufs[p], out_hbm.at[pl.ds(w * _RPW + g * _KG, _KG)], osems[p])

        def o_drain(g):
            p = g % 2
            pltpu.make_async_copy(
                bufs[p], out_hbm.at[pl.ds(w * _RPW + g * _KG, _KG)],
                osems[p]).wait()

        s_fire(0)
        for g in range(_NG):
            s_drain(g)
            r_fire(g)
            if g + 1 < _NG:
                if g >= 1:
                    o_drain(g - 1)
                s_fire(g + 1)
            r_drain(g)
            o_fire(g)
        o_drain(_NG - 2)
        o_drain(_NG - 1)

    return k(ps_t, pr_t, sidx, ridx)


def _sc_scatter(ne, sidx, ridx):
    mesh = plsc.VectorSubcoreMesh(core_axis_name="c", subcore_axis_name="s")

    @functools.partial(
        pl.kernel,
        out_type=(
            jax.ShapeDtypeStruct((_NC, _N, 16), jnp.float32),
            jax.ShapeDtypeStruct((_NC, _N, 16), jnp.float32),
        ),
        mesh=mesh,
        compiler_params=pltpu.CompilerParams(use_tc_tiling_on_sc=False),
        scratch_types=[
            pltpu.VMEM((_RPW, _CH), jnp.int32),
            pltpu.VMEM((_RPW, _CH), jnp.int32),
            pltpu.VMEM((_KG, _CH, 16), jnp.float32),
            pltpu.VMEM((_KG, _CH, 16), jnp.float32),
            pltpu.VMEM((625, 16), jnp.float32),
            pltpu.VMEM_SHARED((_N, 16), jnp.float32),
            pltpu.VMEM_SHARED((_N, 16), jnp.float32),
            pltpu.SemaphoreType.DMA,
            pltpu.SemaphoreType.DMA,
            pltpu.SemaphoreType.DMA,
            pltpu.SemaphoreType.DMA,
        ],
    )
    def k(ne_hbm, sidx_hbm, ridx_hbm, sagg_hbm, ragg_hbm,
          idxs_v, idxr_v, buf_a, buf_b, zbuf, shr_s, shr_r,
          sem_a, sem_b, sem_ca, sem_cb):
        cid = lax.axis_index("c")
        sid = lax.axis_index("s")
        w = cid * _NS + sid

        def zbody(i, _):
            zbuf[i] = jnp.zeros((16,), jnp.float32)
            return 0
        lax.fori_loop(0, 625, zbody, 0)

        pltpu.sync_copy(zbuf, shr_s.at[pl.ds(sid * 625, 625)])
        pltpu.sync_copy(zbuf, shr_r.at[pl.ds(sid * 625, 625)])

        plsc.subcore_barrier()

        pltpu.sync_copy(sidx_hbm.at[pl.ds(w * _RPW, _RPW)], idxs_v)
        pltpu.sync_copy(ridx_hbm.at[pl.ds(w * _RPW, _RPW)], idxr_v)
        bufs, sems = (buf_a, buf_b), (sem_a, sem_b)
        csems = (sem_ca, sem_cb)

        def load_fire(g):
            p = g % 2
            pltpu.async_copy(ne_hbm.at[pl.ds(w * _RPW + g * _KG, _KG)],
                             bufs[p], sems[p])

        def load_drain(g):
            p = g % 2
            pltpu.make_async_copy(
                ne_hbm.at[pl.ds(w * _RPW + g * _KG, _KG)],
                bufs[p], sems[p]).wait()

        def sc_fire(g):
            p = g % 2
            for j in range(_KG):
                pltpu.async_copy(bufs[p].at[j],
                                 shr_s.at[idxs_v.at[g * _KG + j]], csems[p],
                                 add=True)
                pltpu.async_copy(bufs[p].at[j],
                                 shr_r.at[idxr_v.at[g * _KG + j]], csems[p],
                                 add=True)

        def sc_drain(g):
            p = g % 2
            for j in range(_KG):
                pltpu.make_async_copy(
                    bufs[p].at[j], shr_s.at[idxs_v.at[g * _KG + j]],
                    csems[p]).wait()
                pltpu.make_async_copy(
                    bufs[p].at[j], shr_r.at[idxr_v.at[g * _KG + j]],
                    csems[p]).wait()

        load_fire(0)
        for g in range(_NG):
            if g + 1 < _NG:
                if g >= 1:
                    sc_drain(g - 1)
                load_fire(g + 1)
            load_drain(g)
            sc_fire(g)
        sc_drain(_NG - 2)
        sc_drain(_NG - 1)

        plsc.subcore_barrier()

        pltpu.sync_copy(shr_s.at[pl.ds(sid * 625, 625)],
                        sagg_hbm.at[cid, pl.ds(sid * 625, 625), :])
        pltpu.sync_copy(shr_r.at[pl.ds(sid * 625, 625)],
                        ragg_hbm.at[cid, pl.ds(sid * 625, 625), :])

    return k(ne, sidx, ridx)


def _tc_proj(nodes, ws, wr):
    bn = 1000

    def body(n_ref, ws_ref, wr_ref, ps_ref, pr_ref):
        x = n_ref[...]
        ps_ref[...] = jnp.dot(x, ws_ref[...], preferred_element_type=jnp.float32)
        pr_ref[...] = jnp.dot(x, wr_ref[...], preferred_element_type=jnp.float32)

    d = nodes.shape[1]
    return pl.pallas_call(
        body,
        grid=(_N // bn,),
        in_specs=[
            pl.BlockSpec((bn, d), lambda i: (i, 0)),
            pl.BlockSpec((d, 16), lambda i: (0, 0)),
            pl.BlockSpec((d, 16), lambda i: (0, 0)),
        ],
        out_specs=[
            pl.BlockSpec((bn, 16), lambda i: (i, 0)),
            pl.BlockSpec((bn, 16), lambda i: (i, 0)),
        ],
        out_shape=[
            jax.ShapeDtypeStruct((_N, 16), jnp.float32),
            jax.ShapeDtypeStruct((_N, 16), jnp.float32),
        ],
    )(nodes, ws, wr)


def _tc_edge(edges_p, pre, bd1, bd2, c1t, b2t):
    br = 4096

    def body(e_ref, p_ref, bd1_ref, bd2_ref, c1_ref, b2_ref,
             ne_ref, agg_ref):
        i = pl.program_id(0)
        x = jnp.dot(e_ref[...], bd1_ref[...], preferred_element_type=jnp.float32)
        x = x + p_ref[...] + c1_ref[...]
        x = x * jax.nn.sigmoid(x)
        y = jnp.dot(x, bd2_ref[...], preferred_element_type=jnp.float32)
        y = y + b2_ref[...]
        rows = i * br + lax.broadcasted_iota(jnp.int32, (br, 128), 0)
        y = jnp.where(rows < _E8, y, 0.0)
        ne_ref[...] = y

        @pl.when(i == 0)
        def _():
            agg_ref[...] = jnp.zeros_like(agg_ref)

        agg_ref[...] += jnp.sum(y, axis=0, keepdims=True)

    return pl.pallas_call(
        body,
        grid=(_EPK // br,),
        in_specs=[
            pl.BlockSpec((br, 128), lambda i: (i, 0)),
            pl.BlockSpec((br, 128), lambda i: (i, 0)),
            pl.BlockSpec((128, 128), lambda i: (0, 0)),
            pl.BlockSpec((128, 128), lambda i: (0, 0)),
            pl.BlockSpec((1, 128), lambda i: (0, 0)),
            pl.BlockSpec((1, 128), lambda i: (0, 0)),
        ],
        out_specs=[
            pl.BlockSpec((br, 128), lambda i: (i, 0)),
            pl.BlockSpec((1, 128), lambda i: (0, 0)),
        ],
        out_shape=[
            jax.ShapeDtypeStruct((_EPK, 128), jnp.float32),
            jax.ShapeDtypeStruct((1, 128), jnp.float32),
        ],
    )(edges_p, pre, bd1, bd2, c1t, b2t)


def _tc_node(nodes, sagg2, ragg2, wn_n, wn_s, wn_r, cn, wn2, bn2, wsn, wrn):
    bn = 2000
    d = nodes.shape[1]

    def body(n_ref, s_ref, r_ref, wnn_ref, wns_ref, wnr_ref, cn_ref,
             wn2_ref, bn2_ref, wsn_ref, wrn_ref,
             nn_ref, ps_ref, pr_ref, agg_ref):
        i = pl.program_id(0)
        sa = s_ref[0] + s_ref[1]
        ra = r_ref[0] + r_ref[1]
        h = (jnp.dot(n_ref[...], wnn_ref[...], preferred_element_type=jnp.float32)
             + jnp.dot(sa, wns_ref[...], preferred_element_type=jnp.float32)
             + jnp.dot(ra, wnr_ref[...], preferred_element_type=jnp.float32)
             + cn_ref[...])
        h = h * jax.nn.sigmoid(h)
        nn = jnp.dot(h, wn2_ref[...], preferred_element_type=jnp.float32)
        nn = nn + bn2_ref[...]
        nn_ref[...] = nn
        ps_ref[...] = jnp.dot(nn, wsn_ref[...], preferred_element_type=jnp.float32)
        pr_ref[...] = jnp.dot(nn, wrn_ref[...], preferred_element_type=jnp.float32)

        @pl.when(i == 0)
        def _():
            agg_ref[...] = jnp.zeros_like(agg_ref)

        agg_ref[...] += jnp.sum(nn, axis=0, keepdims=True)

    return pl.pallas_call(
        body,
        grid=(_N // bn,),
        in_specs=[
            pl.BlockSpec((bn, d), lambda i: (i, 0)),
            pl.BlockSpec((_NC, bn, 16), lambda i: (0, i, 0)),
            pl.BlockSpec((_NC, bn, 16), lambda i: (0, i, 0)),
            pl.BlockSpec((d, 16), lambda i: (0, 0)),
            pl.BlockSpec((16, 16), lambda i: (0, 0)),
            pl.BlockSpec((16, 16), lambda i: (0, 0)),
            pl.BlockSpec((1, 16), lambda i: (0, 0)),
            pl.BlockSpec((16, 16), lambda i: (0, 0)),
            pl.BlockSpec((1, 16), lambda i: (0, 0)),
            pl.BlockSpec((16, 16), lambda i: (0, 0)),
            pl.BlockSpec((16, 16), lambda i: (0, 0)),
        ],
        out_specs=[
            pl.BlockSpec((bn, 16), lambda i: (i, 0)),
            pl.BlockSpec((bn, 16), lambda i: (i, 0)),
            pl.BlockSpec((bn, 16), lambda i: (i, 0)),
            pl.BlockSpec((1, 16), lambda i: (0, 0)),
        ],
        out_shape=[
            jax.ShapeDtypeStruct((_N, 16), jnp.float32),
            jax.ShapeDtypeStruct((_N, 16), jnp.float32),
            jax.ShapeDtypeStruct((_N, 16), jnp.float32),
            jax.ShapeDtypeStruct((1, 16), jnp.float32),
        ],
    )(nodes, sagg2, ragg2, wn_n, wn_s, wn_r, cn, wn2, bn2, wsn, wrn)


def _tc_node_packed(nodes_pk, sagg_pk, ragg_pk, bdn, bds, bdr, cnt,
                    bd2, bn2t, bdsn, bdrn):
    npk = _N // 8

    def body(n_ref, s_ref, r_ref, bdn_ref, bds_ref, bdr_ref, cn_ref,
             bd2_ref, bn2_ref, bdsn_ref, bdrn_ref,
             nn_ref, ps_ref, pr_ref, agg_ref):
        sa = s_ref[0] + s_ref[1]
        ra = r_ref[0] + r_ref[1]
        h = (jnp.dot(n_ref[...], bdn_ref[...], preferred_element_type=jnp.float32)
             + jnp.dot(sa, bds_ref[...], preferred_element_type=jnp.float32)
             + jnp.dot(ra, bdr_ref[...], preferred_element_type=jnp.float32)
             + cn_ref[...])
        h = h * jax.nn.sigmoid(h)
        nn = jnp.dot(h, bd2_ref[...], preferred_element_type=jnp.float32)
        nn = nn + bn2_ref[...]
        nn_ref[...] = nn
        ps_ref[...] = jnp.dot(nn, bdsn_ref[...], preferred_element_type=jnp.float32)
        pr_ref[...] = jnp.dot(nn, bdrn_ref[...], preferred_element_type=jnp.float32)
        agg_ref[...] = jnp.sum(nn, axis=0, keepdims=True)

    full = lambda s: pl.BlockSpec(s, lambda: tuple(0 for _ in s))
    return pl.pallas_call(
        body,
        in_specs=[
            full((npk, 128)),
            full((_NC, npk, 128)),
            full((_NC, npk, 128)),
            full((128, 128)), full((128, 128)), full((128, 128)),
            full((1, 128)),
            full((128, 128)), full((1, 128)),
            full((128, 128)), full((128, 128)),
        ],
        out_specs=[
            full((npk, 128)), full((npk, 128)), full((npk, 128)),
            full((1, 128)),
        ],
        out_shape=[
            jax.ShapeDtypeStruct((npk, 128), jnp.float32),
            jax.ShapeDtypeStruct((npk, 128), jnp.float32),
            jax.ShapeDtypeStruct((npk, 128), jnp.float32),
            jax.ShapeDtypeStruct((1, 128), jnp.float32),
        ],
    )(nodes_pk, sagg_pk, ragg_pk, bdn, bds, bdr, cnt, bd2, bn2t, bdsn, bdrn)


def _kron8(w):
    return jnp.kron(jnp.eye(8, dtype=jnp.float32), w)


def _tile8(v):
    return jnp.tile(v.reshape(1, 16), (1, 8))


def kernel(nodes, edges, senders, receivers, params):
    blocks = ([params['encoder']] + [params['processor']] * 4
              + [params['decoder']])

    pad = _EP - _E
    sidx = jnp.concatenate(
        [senders, jnp.zeros((pad,), jnp.int32)]).reshape(_NRI, _CH)
    ridx = jnp.concatenate(
        [receivers, jnp.zeros((pad,), jnp.int32)]).reshape(_NRI, _CH)
    edges_p = edges.reshape(_E8, 128)

    g = jnp.zeros((1, 1), jnp.float32)
    nodes_cur = nodes
    zero16 = jnp.zeros((16, 16), jnp.float32)

    def _esplit(b, d_n, d_g):
        (w1, b1), (w2, b2) = b['edge']
        return (w1[:16], w1[16:16 + d_n], w1[16 + d_n:16 + 2 * d_n],
                w1[-d_g:], b1, w2, b2)

    we1_e0, ws0, wr0, _, _, _, _ = _esplit(blocks[0], 128, 1)
    ps, pr = _tc_proj(nodes, ws0, wr0)

    for bi, blk in enumerate(blocks):
        d_n = 128 if bi == 0 else 16
        d_g = g.shape[1]
        we1_e, _, _, we1_g, be1, we2, be2 = _esplit(blk, d_n, d_g)
        c1 = g @ we1_g + be1.reshape(1, 16)

        pre = _sc_gather(ps, pr, sidx, ridx)
        new_edges, eagg_p = _tc_edge(
            edges_p, pre.reshape(_EPK, 128),
            _kron8(we1_e), _kron8(we2), _tile8(c1), _tile8(be2.reshape(1, 16)))
        edge_agg = eagg_p.reshape(8, 16).sum(axis=0, keepdims=True)

        sagg2, ragg2 = _sc_scatter(new_edges.reshape(_NRI, _CH, 16), sidx, ridx)

        (wn1, bn1), (wn2, bn2) = blk['node']
        wn_n = wn1[:d_n]
        wn_s = wn1[d_n:d_n + 16]
        wn_r = wn1[d_n + 16:d_n + 32]
        wn_g = wn1[-d_g:]
        cn = g @ wn_g + bn1.reshape(1, 16)

        if bi + 1 < len(blocks):
            nblk = blocks[bi + 1]
            wsn = nblk['edge'][0][0][16:32]
            wrn = nblk['edge'][0][0][32:48]
        else:
            wsn = wrn = zero16

        if bi == 0:
            new_nodes, ps, pr, node_agg = _tc_node(
                nodes_cur, sagg2, ragg2, wn_n, wn_s, wn_r, cn,
                wn2, bn2.reshape(1, 16), wsn, wrn)
            nodes_cur = new_nodes.reshape(_N // 8, 128)
        else:
            nodes_cur, ps_pk, pr_pk, nagg_p = _tc_node_packed(
                nodes_cur, sagg2.reshape(_NC, _N // 8, 128),
                ragg2.reshape(_NC, _N // 8, 128),
                _kron8(wn_n), _kron8(wn_s), _kron8(wn_r), _tile8(cn),
                _kron8(wn2), _tile8(bn2.reshape(1, 16)),
                _kron8(wsn), _kron8(wrn))
            ps = ps_pk.reshape(_N, 16)
            pr = pr_pk.reshape(_N, 16)
            node_agg = nagg_p.reshape(8, 16).sum(axis=0, keepdims=True)

        (wg1, bg1), (wg2, bg2) = blk['global']
        cat = jnp.concatenate([node_agg, edge_agg, g], axis=1)
        hg = cat @ wg1 + bg1.reshape(1, -1)
        hg = hg * jax.nn.sigmoid(hg)
        g = hg @ wg2 + bg2.reshape(1, -1)

        edges_p = new_edges

    return g[:, 0]
